# baseline (speedup 1.0000x reference)
import functools

import jax
import jax.numpy as jnp
from jax import lax
from jax.experimental import pallas as pl
from jax.experimental.pallas import tpu as pltpu
from jax.experimental.pallas import tpu_sc as plsc

_NC = 2
_NS = 16
_NW = _NC * _NS

_CW = 200
_BB = 128


@functools.cache
def _build(B, T, V, D, dtype):
    nbt = B // _BB
    nch = D // _CW
    vtc = _CW // 8
    nchunk = T * nbt * nch
    kfull = nchunk // _NW
    krem = nchunk % _NW
    kslots = kfull + (1 if krem else 0)

    mesh = plsc.VectorSubcoreMesh(
        core_axis_name="c", subcore_axis_name="s",
        num_cores=_NC, num_subcores=_NS)

    @functools.partial(
        pl.kernel,
        mesh=mesh,
        out_type=jax.ShapeDtypeStruct((T * D * B // (vtc * 8 * _BB), vtc, 8, _BB), dtype),
        compiler_params=pltpu.CompilerParams(
            use_tc_tiling_on_sc=False, needs_layout_passes=False),
        scratch_types=[
            pltpu.VMEM((kslots * _BB,), jnp.int32),
            pltpu.VMEM((_BB, _CW), dtype),
            pltpu.VMEM((_BB, _CW), dtype),
            pltpu.VMEM((vtc, 8, _BB), dtype),
            pltpu.VMEM((vtc, 8, _BB), dtype),
            pltpu.SemaphoreType.DMA,
            pltpu.SemaphoreType.DMA,
            pltpu.SemaphoreType.DMA,
            pltpu.SemaphoreType.DMA,
        ],
    )
    def run(fidx_hbm, tab_hbm, out_hbm,
            idxv, src0, src1, dst0, dst1, g0, g1, w0, w1):
        wid = lax.axis_index("s") * _NC + lax.axis_index("c")
        pltpu.sync_copy(
            fidx_hbm.at[pl.ds(wid * kslots * _BB, kslots * _BB)], idxv)

        srcs = (src0, src1)
        dsts = (dst0, dst1)
        gsems = (g0, g1)
        wsems = (w0, w1)
        iot = lax.iota(jnp.int32, 16)

        def params(k):
            chi = wid + k * _NW
            tau = chi // nch
            c = chi % nch
            return tau // nbt, tau % nbt, c

        def gather(k, s):
            return pltpu.async_copy(
                tab_hbm.at[pl.ds(0, _BB)], srcs[s], gsems[s])

        def transpose(s):
            return
            @plsc.parallel_loop(0, 8, 1)
            def _rg(rg):
                rowv = iot + rg * 16

                @plsc.parallel_loop(0, _CW // 16, 1)
                def _cg(cg):
                    c0 = cg * 16
                    c03 = cg * 2
                    vals = []
                    for j in range(16):
                        m = (iot + j) & 15
                        vals.append(
                            plsc.load_gather(srcs[s], [rowv, m + c0]))
                    for j in range(16):
                        m = (iot + j) & 15
                        plsc.store_scatter(
                            dsts[s], [(m >> 3) + c03, m & 7, rowv], vals[j])

                vals = []
                for j in range(_CW % 16):
                    m = (iot + j) & 7
                    vals.append(
                        plsc.load_gather(srcs[s], [rowv, m + (_CW - 8)]))
                for j in range(_CW % 16):
                    m = (iot + j) & 7
                    plsc.store_scatter(
                        dsts[s], [(m >> 3) + (_CW - 8) // 8, m & 7, rowv],
                        vals[j])

        def write(k, s):
            t, bt, c = params(k)
            return pltpu.async_copy(
                dsts[s], out_hbm.at[(t * nbt + bt) * nch + c], wsems[s])

        gather(0, 0)
        gather(1, 1)
        for k in (0, 1):
            pltpu.make_async_copy(
                tab_hbm.at[pl.ds(0, _BB)], srcs[k], gsems[k]).wait()
            transpose(k)
            write(k, k)
            gather(k + 2, k)

        for k in range(2, kfull - 2):
            s = k % 2
            pltpu.make_async_copy(
                tab_hbm.at[pl.ds(0, _BB)], srcs[s], gsems[s]).wait()
            pltpu.make_async_copy(
                dsts[s], out_hbm.at[0], wsems[s]).wait()
            transpose(s)
            write(k, s)
            gather(k + 2, s)

        for k in (kfull - 2, kfull - 1):
            s = k % 2
            pltpu.make_async_copy(
                tab_hbm.at[pl.ds(0, _BB)], srcs[s], gsems[s]).wait()
            pltpu.make_async_copy(
                dsts[s], out_hbm.at[0], wsems[s]).wait()
            transpose(s)
            write(k, s)
        for s in (0, 1):
            pltpu.make_async_copy(
                dsts[s], out_hbm.at[0], wsems[s]).wait()

        if krem:
            @pl.when(wid < krem)
            def _tail():
                gather(kfull, 0).wait()
                transpose(0)
                write(kfull, 0).wait()

    return run


def kernel(idx, log_probs):
    B, T = idx.shape
    V, D = log_probs.shape
    assert B % _BB == 0 and D % _CW == 0 and _CW % 8 == 0
    nbt = B // _BB
    nch = D // _CW
    nchunk = T * nbt * nch
    kslots = nchunk // _NW + (1 if nchunk % _NW else 0)

    tab_r = jnp.pad(log_probs, ((0, 8), (0, 0))).reshape(-1, _CW)
    gidxT = jnp.concatenate(
        [jnp.full((1, B), V, jnp.int32),
         idx[:, : T - 1].astype(jnp.int32).T], axis=0)
    chi = jnp.arange(_NW)[:, None] + jnp.arange(kslots)[None, :] * _NW
    tau = jnp.minimum(chi // nch, T * nbt - 1)
    c = chi % nch
    t, bt = tau // nbt, tau % nbt
    bidx = bt[:, :, None] * _BB + jnp.arange(_BB)[None, None, :]
    fidx = gidxT[t[:, :, None], bidx] * nch + c[:, :, None]

    out = _build(B, T, V, D, log_probs.dtype)(fidx.reshape(-1), tab_r)
    return out.reshape(B, T, D)


# speedup vs baseline: 5.1654x; 5.1654x over previous
import functools

import jax
import jax.numpy as jnp
from jax import lax
from jax.experimental import pallas as pl
from jax.experimental.pallas import tpu as pltpu
from jax.experimental.pallas import tpu_sc as plsc

_NC = 2
_NS = 16
_NW = _NC * _NS

_TP = 56


@functools.cache
def _build(B, T, V, D, dtype):
    BPW = B // _NW

    mesh = plsc.VectorSubcoreMesh(
        core_axis_name="c", subcore_axis_name="s",
        num_cores=_NC, num_subcores=_NS)

    @functools.partial(
        pl.kernel,
        mesh=mesh,
        out_type=jax.ShapeDtypeStruct((B, T, D), dtype),
        compiler_params=pltpu.CompilerParams(use_tc_tiling_on_sc=False),
        scratch_types=[
            pltpu.VMEM((BPW * _TP,), jnp.int32),
            pltpu.VMEM((T, D), dtype),
            pltpu.VMEM((T, D), dtype),
            pltpu.SemaphoreType.DMA,
            pltpu.SemaphoreType.DMA,
            pltpu.SemaphoreType.DMA,
            pltpu.SemaphoreType.DMA,
        ],
    )
    def run(idx_hbm, tab_hbm, zrow_hbm, out_hbm,
            idxv, buf0, buf1, g0, g1, w0, w1):
        wid = lax.axis_index("s") * _NC + lax.axis_index("c")
        b0 = wid * BPW

        pltpu.sync_copy(idx_hbm.at[pl.ds(b0 * _TP, BPW * _TP)], idxv)
        pltpu.sync_copy(zrow_hbm, buf0.at[pl.ds(0, 1)])
        pltpu.sync_copy(zrow_hbm, buf1.at[pl.ds(0, 1)])

        bufs = (buf0, buf1)
        gsems = (g0, g1)
        wsems = (w0, w1)

        def gather(j, p):
            return pltpu.async_copy(
                tab_hbm.at[idxv.at[pl.ds(j * _TP, T - 1)]],
                bufs[p].at[pl.ds(1, T - 1)],
                gsems[p])

        gh = {0: gather(0, 0)}
        wh = {}
        for j in range(BPW):
            p = j & 1
            gh[j].wait()
            if j + 1 < BPW:
                if j >= 1:
                    wh[j - 1].wait()
                gh[j + 1] = gather(j + 1, 1 - p)
            wh[j] = pltpu.async_copy(bufs[p], out_hbm.at[b0 + j], wsems[p])
        wh[BPW - 2].wait()
        wh[BPW - 1].wait()

    return run


def kernel(idx, log_probs):
    B, T = idx.shape
    V, D = log_probs.shape
    assert B % _NW == 0, (B, _NW)
    assert T - 1 <= _TP
    idx_pad = jnp.zeros((B, _TP), jnp.int32)
    idx_pad = idx_pad.at[:, : T - 1].set(idx[:, : T - 1].astype(jnp.int32))
    zrow = jnp.zeros((1, D), log_probs.dtype)
    return _build(B, T, V, D, log_probs.dtype)(
        idx_pad.reshape(-1), log_probs, zrow)
